# Initial kernel scaffold; baseline (speedup 1.0000x reference)
import jax
import jax.numpy as jnp
from jax.experimental import pallas as pl


def kernel(x, edge_index, W1, b1, Wp, bp):
    raise NotImplementedError("write your pallas kernel here")


# speedup vs baseline: 19.2234x; 19.2234x over previous
import functools

import jax
import jax.numpy as jnp
from jax import lax
from jax.experimental import pallas as pl
from jax.experimental.pallas import tpu as pltpu
from jax.experimental.pallas import tpu_sc as plsc

NC, NS = 2, 16
NW = NC * NS
CHUNK = 128
RB = 1000


@functools.lru_cache(maxsize=None)
def _hist_kernel(n_pad, nch, rows_z):
    mesh = plsc.VectorSubcoreMesh(core_axis_name="c", subcore_axis_name="s")

    @functools.partial(
        pl.kernel,
        mesh=mesh,
        out_type=jax.ShapeDtypeStruct((NC, n_pad, 16), jnp.float32),
        scratch_types=[
            pltpu.VMEM((nch, CHUNK), jnp.int32),
            pltpu.VMEM((CHUNK, 16), jnp.float32),
            pltpu.VMEM_SHARED((n_pad, 16), jnp.float32),
        ],
    )
    def k(dst_hbm, zeros_hbm, ones_hbm, out_hbm, idx_v, ones_v, acc):
        c = lax.axis_index("c")
        s = lax.axis_index("s")
        wid = c * NS + s
        pltpu.sync_copy(zeros_hbm.at[pl.ds(s * rows_z, rows_z)],
                        acc.at[pl.ds(s * rows_z, rows_z)])
        pltpu.sync_copy(ones_hbm, ones_v)
        pltpu.sync_copy(dst_hbm.at[wid], idx_v)
        plsc.subcore_barrier()

        def body(j, carry):
            pltpu.sync_copy(ones_v, acc.at[idx_v.at[j]], add=True)
            return carry

        lax.fori_loop(0, nch, body, 0)
        plsc.subcore_barrier()
        pltpu.sync_copy(acc.at[pl.ds(s * rows_z, rows_z)],
                        out_hbm.at[c, pl.ds(s * rows_z, rows_z)])

    return k


@functools.lru_cache(maxsize=None)
def _scatter_kernel(n_pad, nch, rows_z, d):
    mesh = plsc.VectorSubcoreMesh(core_axis_name="c", subcore_axis_name="s")

    @functools.partial(
        pl.kernel,
        mesh=mesh,
        out_type=jax.ShapeDtypeStruct((NC, n_pad, d), jnp.float32),
        scratch_types=[
            pltpu.VMEM((nch, CHUNK), jnp.int32),
            pltpu.VMEM((nch, CHUNK), jnp.int32),
            pltpu.VMEM((CHUNK, d), jnp.float32),
            pltpu.VMEM_SHARED((n_pad, d), jnp.float32),
            pltpu.SemaphoreType.DMA,
        ],
    )
    def k(h2_hbm, src_hbm, dst_hbm, zeros_hbm, out_hbm,
          src_v, dst_v, rows_v, acc, sem):
        c = lax.axis_index("c")
        s = lax.axis_index("s")
        wid = c * NS + s
        pltpu.sync_copy(zeros_hbm.at[pl.ds(s * rows_z, rows_z)],
                        acc.at[pl.ds(s * rows_z, rows_z)])
        pltpu.sync_copy(src_hbm.at[wid], src_v)
        pltpu.sync_copy(dst_hbm.at[wid], dst_v)
        plsc.subcore_barrier()

        def body(j, carry):
            pltpu.async_copy(h2_hbm.at[src_v.at[j]], rows_v, sem).wait()
            pltpu.sync_copy(rows_v, acc.at[dst_v.at[j]], add=True)
            return carry

        lax.fori_loop(0, nch, body, 0)
        plsc.subcore_barrier()
        pltpu.sync_copy(acc.at[pl.ds(s * rows_z, rows_z)],
                        out_hbm.at[c, pl.ds(s * rows_z, rows_z)])

    return k


def _tc1_body(x_ref, w1_ref, c0_ref, c1_ref, h2_ref, dis_ref):
    cnt = c0_ref[:, 0:1] + c1_ref[:, 0:1]
    dis = lax.rsqrt(cnt + 1.0)
    h = jnp.dot(x_ref[...], w1_ref[...], preferred_element_type=jnp.float32)
    h2_ref[...] = h * dis
    dis_ref[...] = dis


def _tc1(x, w1, c0, c1):
    n, d_in = x.shape
    d_hid = w1.shape[1]
    grid = n // RB
    return pl.pallas_call(
        _tc1_body,
        grid=(grid,),
        in_specs=[
            pl.BlockSpec((RB, d_in), lambda i: (i, 0)),
            pl.BlockSpec((d_in, d_hid), lambda i: (0, 0)),
            pl.BlockSpec((RB, 16), lambda i: (i, 0)),
            pl.BlockSpec((RB, 16), lambda i: (i, 0)),
        ],
        out_specs=[
            pl.BlockSpec((RB, d_hid), lambda i: (i, 0)),
            pl.BlockSpec((RB, 1), lambda i: (i, 0)),
        ],
        out_shape=[
            jax.ShapeDtypeStruct((n, d_hid), jnp.float32),
            jax.ShapeDtypeStruct((n, 1), jnp.float32),
        ],
    )(x, w1, c0, c1)


def _tc2_body(a0_ref, a1_ref, h2_ref, dis_ref, b1_ref, wp_ref, bp_ref, out_ref):
    t = (a0_ref[...] + a1_ref[...] + h2_ref[...]) * dis_ref[...]
    t = jnp.maximum(t + b1_ref[...], 0.0)
    out_ref[...] = (
        jnp.dot(t, wp_ref[...], preferred_element_type=jnp.float32) + bp_ref[...]
    )


def _tc2(a0, a1, h2, dis, b1, wp, bp):
    n, d_hid = h2.shape
    d_out = wp.shape[1]
    grid = n // RB
    return pl.pallas_call(
        _tc2_body,
        grid=(grid,),
        in_specs=[
            pl.BlockSpec((RB, d_hid), lambda i: (i, 0)),
            pl.BlockSpec((RB, d_hid), lambda i: (i, 0)),
            pl.BlockSpec((RB, d_hid), lambda i: (i, 0)),
            pl.BlockSpec((RB, 1), lambda i: (i, 0)),
            pl.BlockSpec((1, d_hid), lambda i: (0, 0)),
            pl.BlockSpec((d_hid, d_out), lambda i: (0, 0)),
            pl.BlockSpec((1, d_out), lambda i: (0, 0)),
        ],
        out_specs=pl.BlockSpec((RB, d_out), lambda i: (i, 0)),
        out_shape=jax.ShapeDtypeStruct((n, d_out), jnp.float32),
    )(a0, a1, h2, dis, b1, wp, bp)


def kernel(x, edge_index, W1, b1, Wp, bp):
    n, d_in = x.shape
    d = W1.shape[1]
    e = edge_index.shape[1]

    n_pad = ((n + 1 + 127) // 128) * 128
    rows_z = n_pad // NS
    nch = -(-e // (NW * CHUNK))
    e_pad = NW * nch * CHUNK

    src = edge_index[0].astype(jnp.int32)
    dst = edge_index[1].astype(jnp.int32)
    pad = e_pad - e
    src_p = jnp.concatenate([src, jnp.zeros((pad,), jnp.int32)]).reshape(
        NW, nch, CHUNK)
    dst_p = jnp.concatenate([dst, jnp.full((pad,), n, jnp.int32)]).reshape(
        NW, nch, CHUNK)

    zeros16 = jnp.zeros((n_pad, 16), jnp.float32)
    ones16 = jnp.ones((CHUNK, 16), jnp.float32)
    zeros_d = jnp.zeros((n_pad, d), jnp.float32)

    counts = _hist_kernel(n_pad, nch, rows_z)(dst_p, zeros16, ones16)
    h2, dis = _tc1(x, W1, counts[0, :n], counts[1, :n])
    acc = _scatter_kernel(n_pad, nch, rows_z, d)(h2, src_p, dst_p, zeros_d)
    out = _tc2(acc[0, :n], acc[1, :n], h2, dis,
               b1.reshape(1, d), Wp, bp.reshape(1, -1))
    return out
